# Initial kernel scaffold; baseline (speedup 1.0000x reference)
import jax
import jax.numpy as jnp
from jax.experimental import pallas as pl


def kernel(x, weight):
    raise NotImplementedError("write your pallas kernel here")


# speedup vs baseline: 3.0269x; 3.0269x over previous
import jax
import jax.numpy as jnp
from jax import lax
from jax.experimental import pallas as pl
from jax.experimental.pallas import tpu as pltpu
from jax.experimental.pallas import tpu_sc as plsc

VOCAB = 100000
D_MODEL = 512
BATCH = 1024
SEQ = 50
SEQ_PAD = 64

_NC = 2
_NS = 16
_NW = _NC * _NS
_SEQ_PER_W = BATCH // _NW
_LANES = 16
_DCHUNKS = D_MODEL // _LANES


def _pe_table():
    pos = jnp.arange(SEQ, dtype=jnp.float32)[:, None]
    hid = jnp.arange(D_MODEL, dtype=jnp.float32)[None, :]
    angle = pos / jnp.power(10000.0, 2.0 * jnp.floor(hid / 2.0) / D_MODEL)
    even = (jnp.arange(SEQ) % 2 == 0)[:, None]
    return jnp.where(even, jnp.sin(angle), jnp.cos(angle))


def _body(x_hbm, w_hbm, pe_hbm, out_hbm, pe_v, idx_v, buf, mask_v, gsem):
    wid = lax.axis_index("s") * _NC + lax.axis_index("c")
    seq0 = wid * _SEQ_PER_W

    pltpu.sync_copy(pe_hbm, pe_v)
    pltpu.sync_copy(x_hbm.at[pl.ds(seq0, _SEQ_PER_W), :], idx_v)

    def per_seq(c, carry):
        pltpu.async_copy(
            w_hbm.at[idx_v.at[c, pl.ds(0, SEQ)]], buf, gsem
        ).wait()

        for k in range(SEQ_PAD // _LANES):
            ksl = pl.ds(k * _LANES, _LANES)
            mask_v[ksl] = jnp.minimum(idx_v[c, ksl], 1).astype(jnp.float32)

        def per_row(r, carry2):
            splat = jnp.full((_LANES,), r, dtype=jnp.int32)
            mv = plsc.load_gather(mask_v, [splat])
            for d in range(_DCHUNKS):
                sl = pl.ds(d * _LANES, _LANES)
                buf[r, sl] = buf[r, sl] * mv + pe_v[r, sl]
            return carry2

        lax.fori_loop(0, SEQ, per_row, 0, unroll=1)
        pltpu.sync_copy(buf, out_hbm.at[seq0 + c])
        return carry

    lax.fori_loop(0, _SEQ_PER_W, per_seq, 0, unroll=1)


@jax.jit
def _embed(xp, w, pe):
    mesh = plsc.VectorSubcoreMesh(core_axis_name="c", subcore_axis_name="s")
    f = pl.kernel(
        _body,
        out_type=jax.ShapeDtypeStruct((BATCH, SEQ, D_MODEL), jnp.float32),
        mesh=mesh,
        compiler_params=pltpu.CompilerParams(needs_layout_passes=False),
        scratch_types=[
            pltpu.VMEM((SEQ, D_MODEL), jnp.float32),
            pltpu.VMEM((_SEQ_PER_W, SEQ_PAD), jnp.int32),
            pltpu.VMEM((SEQ, D_MODEL), jnp.float32),
            pltpu.VMEM((SEQ_PAD,), jnp.float32),
            pltpu.SemaphoreType.DMA,
        ],
    )
    return f(xp, w, pe)


def kernel(x, weight):
    pe = _pe_table()
    xp = jnp.pad(
        x.astype(jnp.int32), ((0, 0), (0, SEQ_PAD - SEQ)), constant_values=1
    )
    return _embed(xp, weight, pe)
